# Initial kernel scaffold; baseline (speedup 1.0000x reference)
import jax
import jax.numpy as jnp
from jax.experimental import pallas as pl


def kernel(pos, x, batch, W1_0, b1_0, W2_0, b2_0, g_0, be_0, W1_1, b1_1, W2_1, b2_1, g_1, be_1):
    raise NotImplementedError("write your pallas kernel here")


# speedup vs baseline: 16.9930x; 16.9930x over previous
import functools

import jax
import jax.numpy as jnp
from jax import lax
from jax.experimental import pallas as pl
from jax.experimental.pallas import tpu as pltpu
from jax.experimental.pallas import tpu_sc as plsc

N_ATOM_TYPES = 16
GRID_SIZE = 8
CODE_DIM = 128
HIDDEN_DIM = 256
K_ATOM = 8
K_GRID = 32
B = 4
N_PER = 1024
N = B * N_PER
N_GRID = GRID_SIZE ** 3
NG = B * N_GRID
NT = N + NG
E_ATOM = N * K_ATOM
E_GRID = NG * K_GRID
E_TOT = E_ATOM + E_GRID

NW = 32


def _knn_body(q_ref, k_ref, a2_ref, b2_ref, idx_ref, dist_ref, *, k, nq, diag):
    b = pl.program_id(0)
    q = q_ref[...]
    kp = k_ref[...]
    g = lax.dot_general(q, kp, (((1,), (1,)), ((), ())),
                        preferred_element_type=jnp.float32)
    d2 = (a2_ref[...] + b2_ref[...]) - 2.0 * g
    jj = lax.broadcasted_iota(jnp.int32, (nq, N_PER), 1)
    if diag:
        ii = lax.broadcasted_iota(jnp.int32, (nq, N_PER), 0)
        d2 = jnp.where(ii == jj, jnp.inf, d2)
    for t in range(k):
        m = jnp.min(d2, axis=1, keepdims=True)
        am = jnp.min(jnp.where(d2 <= m, jj, jnp.int32(2**30)),
                     axis=1, keepdims=True)
        idx_ref[:, t:t + 1] = am + b * N_PER
        dist_ref[:, t:t + 1] = jnp.sqrt(jnp.maximum(m, 0.0))
        d2 = jnp.where(jj == am, jnp.inf, d2)


def _knn_atoms(pos, p2col, p2row):
    return pl.pallas_call(
        functools.partial(_knn_body, k=K_ATOM, nq=N_PER, diag=True),
        grid=(B,),
        in_specs=[pl.BlockSpec((N_PER, 3), lambda b: (b, 0)),
                  pl.BlockSpec((N_PER, 3), lambda b: (b, 0)),
                  pl.BlockSpec((N_PER, 1), lambda b: (b, 0)),
                  pl.BlockSpec((1, N_PER), lambda b: (0, b))],
        out_specs=[pl.BlockSpec((N_PER, K_ATOM), lambda b: (b, 0)),
                   pl.BlockSpec((N_PER, K_ATOM), lambda b: (b, 0))],
        out_shape=[jax.ShapeDtypeStruct((N, K_ATOM), jnp.int32),
                   jax.ShapeDtypeStruct((N, K_ATOM), jnp.float32)],
    )(pos, pos, p2col, p2row)


def _knn_grid(gc, pos, g2col, p2row):
    return pl.pallas_call(
        functools.partial(_knn_body, k=K_GRID, nq=N_GRID, diag=False),
        grid=(B,),
        in_specs=[pl.BlockSpec((N_GRID, 3), lambda b: (0, 0)),
                  pl.BlockSpec((N_PER, 3), lambda b: (b, 0)),
                  pl.BlockSpec((N_GRID, 1), lambda b: (0, 0)),
                  pl.BlockSpec((1, N_PER), lambda b: (0, b))],
        out_specs=[pl.BlockSpec((N_GRID, K_GRID), lambda b: (b, 0)),
                   pl.BlockSpec((N_GRID, K_GRID), lambda b: (b, 0))],
        out_shape=[jax.ShapeDtypeStruct((NG, K_GRID), jnp.int32),
                   jax.ShapeDtypeStruct((NG, K_GRID), jnp.float32)],
    )(gc, pos, g2col, p2row)


def _pre_body(h_ref, wa_ref, wb_ref, b1_ref, a_ref, bc_ref):
    h = h_ref[...]
    a_ref[...] = jnp.dot(h, wa_ref[...], preferred_element_type=jnp.float32)
    bc_ref[...] = (jnp.dot(h, wb_ref[...], preferred_element_type=jnp.float32)
                   + b1_ref[...])


def _pre(h, wa, wb, b1):
    rows = h.shape[0]
    blk = 512
    return pl.pallas_call(
        _pre_body,
        grid=(rows // blk,),
        in_specs=[pl.BlockSpec((blk, CODE_DIM), lambda i: (i, 0)),
                  pl.BlockSpec((CODE_DIM, HIDDEN_DIM), lambda i: (0, 0)),
                  pl.BlockSpec((CODE_DIM, HIDDEN_DIM), lambda i: (0, 0)),
                  pl.BlockSpec((1, HIDDEN_DIM), lambda i: (0, 0))],
        out_specs=[pl.BlockSpec((blk, HIDDEN_DIM), lambda i: (i, 0)),
                   pl.BlockSpec((blk, HIDDEN_DIM), lambda i: (i, 0))],
        out_shape=[jax.ShapeDtypeStruct((rows, HIDDEN_DIM), jnp.float32),
                   jax.ShapeDtypeStruct((rows, HIDDEN_DIM), jnp.float32)],
    )(h, wa, wb, b1)


def _post_body(s_ref, w2_ref, b2_ref, h_ref, g_ref, be_ref, o_ref):
    aggr = (jnp.dot(s_ref[...], w2_ref[...], preferred_element_type=jnp.float32)
            + b2_ref[...])
    hn = h_ref[...] + aggr
    mu = jnp.mean(hn, axis=1, keepdims=True)
    cen = hn - mu
    var = jnp.mean(cen * cen, axis=1, keepdims=True)
    o_ref[...] = cen / jnp.sqrt(var + 1e-5) * g_ref[...] + be_ref[...]


def _post(s, w2, b2, h, g, be):
    rows = h.shape[0]
    blk = 512
    return pl.pallas_call(
        _post_body,
        grid=(rows // blk,),
        in_specs=[pl.BlockSpec((blk, HIDDEN_DIM), lambda i: (i, 0)),
                  pl.BlockSpec((HIDDEN_DIM, CODE_DIM), lambda i: (0, 0)),
                  pl.BlockSpec((1, CODE_DIM), lambda i: (0, 0)),
                  pl.BlockSpec((blk, CODE_DIM), lambda i: (i, 0)),
                  pl.BlockSpec((1, CODE_DIM), lambda i: (0, 0)),
                  pl.BlockSpec((1, CODE_DIM), lambda i: (0, 0))],
        out_specs=pl.BlockSpec((blk, CODE_DIM), lambda i: (i, 0)),
        out_shape=jax.ShapeDtypeStruct((rows, CODE_DIM), jnp.float32),
    )(s, w2, b2, h, g, be)


_EPB_A = 128 // K_ATOM
_EPB_G = 128 // K_GRID
_BLKS_A = N // NW // _EPB_A
_BLKS_G = NG // NW // _EPB_G


def _sc_msg_body(include_atoms, a_hbm, bc_hbm, idx_hbm, dist_hbm, w_hbm,
                 s_hbm, w_v, idx_v, dist_v, rows_v, bc_v, s_v, sem):
    cid = lax.axis_index("c")
    sid = lax.axis_index("s")
    wid = sid * 2 + cid

    pltpu.sync_copy(w_hbm, w_v)
    ws = [w_v[pl.ds(i * 16, 16)] for i in range(16)]

    def run_block(e0, c0, nc, k):
        pltpu.sync_copy(idx_hbm.at[pl.ds(e0, 128)], idx_v)
        pltpu.sync_copy(dist_hbm.at[pl.ds(e0, 128)], dist_v.at[pl.ds(0, 128)])
        pltpu.sync_copy(bc_hbm.at[pl.ds(c0, nc)], bc_v.at[pl.ds(0, nc)])
        pltpu.async_copy(a_hbm.at[idx_v], rows_v, sem).wait()
        inv = jnp.float32(1.0 / k)

        def center(ci, _):
            for half in range(2):
                bcs = [bc_v[ci, pl.ds((half * 8 + s) * 16, 16)]
                       for s in range(8)]
                acc = [jnp.zeros((16,), jnp.float32) for _ in range(8)]
                for j in range(k):
                    r = ci * k + j
                    dvec = jnp.full((16,), dist_v[pl.ds(r, 16)][0],
                                    jnp.float32)
                    for s in range(8):
                        t = (rows_v[r, pl.ds((half * 8 + s) * 16, 16)]
                             + bcs[s] + dvec * ws[half * 8 + s])
                        acc[s] = acc[s] + jnp.maximum(t, jnp.float32(0.0))
                for s in range(8):
                    s_v[ci, pl.ds((half * 8 + s) * 16, 16)] = acc[s] * inv
            return _

        lax.fori_loop(0, nc, center, None)
        pltpu.sync_copy(s_v.at[pl.ds(0, nc)], s_hbm.at[pl.ds(c0, nc)])

    if include_atoms:
        def atom_block(bi, _):
            run_block(wid * (N // NW) * K_ATOM + bi * 128,
                      wid * (N // NW) + bi * _EPB_A, _EPB_A, K_ATOM)
            return _
        lax.fori_loop(0, _BLKS_A, atom_block, None)

    def grid_block(bi, _):
        run_block(E_ATOM + wid * (NG // NW) * K_GRID + bi * 128,
                  N + wid * (NG // NW) + bi * _EPB_G, _EPB_G, K_GRID)
        return _
    lax.fori_loop(0, _BLKS_G, grid_block, None)


def _msg(a, bc, idx, dist, w, include_atoms):
    mesh = plsc.VectorSubcoreMesh(core_axis_name="c", subcore_axis_name="s")
    f = pl.kernel(
        functools.partial(_sc_msg_body, include_atoms),
        out_type=jax.ShapeDtypeStruct((NT, HIDDEN_DIM), jnp.float32),
        mesh=mesh,
        scratch_types=[
            pltpu.VMEM((HIDDEN_DIM,), jnp.float32),
            pltpu.VMEM((128,), jnp.int32),
            pltpu.VMEM((144,), jnp.float32),
            pltpu.VMEM((128, HIDDEN_DIM), jnp.float32),
            pltpu.VMEM((_EPB_A, HIDDEN_DIM), jnp.float32),
            pltpu.VMEM((_EPB_A, HIDDEN_DIM), jnp.float32),
            pltpu.SemaphoreType.DMA,
        ],
    )
    return f(a, bc, idx, dist, w)


def kernel(pos, x, batch, W1_0, b1_0, W2_0, b2_0, g_0, be_0,
           W1_1, b1_1, W2_1, b2_1, g_1, be_1):
    f32 = jnp.float32
    pos = pos.astype(f32)

    p2 = jnp.sum(pos * pos, axis=1)
    p2col = p2.reshape(N, 1)
    p2row = p2.reshape(1, N)

    g1d = jnp.linspace(-1.0, 1.0, GRID_SIZE)
    mx, my, mz = jnp.meshgrid(g1d, g1d, g1d, indexing='ij')
    gc = jnp.stack([mx, my, mz], axis=-1).reshape(-1, 3).astype(f32)
    g2col = jnp.sum(gc * gc, axis=1).reshape(N_GRID, 1)

    idxA, distA = _knn_atoms(pos, p2col, p2row)
    idxG, distG = _knn_grid(gc, pos, g2col, p2row)
    idx_flat = jnp.concatenate([idxA.reshape(-1), idxG.reshape(-1)])
    dist_flat = jnp.concatenate([distA.reshape(-1), distG.reshape(-1)])

    h0a = jax.nn.one_hot(x, N_ATOM_TYPES, dtype=f32)
    h = jnp.concatenate(
        [jnp.pad(h0a, ((0, 0), (0, CODE_DIM - N_ATOM_TYPES))),
         jnp.zeros((NG, CODE_DIM), f32)], axis=0)

    params = [(W1_0, b1_0, W2_0, b2_0, g_0, be_0),
              (W1_1, b1_1, W2_1, b2_1, g_1, be_1)]

    W1, b1, W2, b2, g, be = params[0]
    A, Bc = _pre(h, W1[:CODE_DIM], W1[CODE_DIM:2 * CODE_DIM],
                 b1.reshape(1, HIDDEN_DIM))
    S = _msg(A, Bc, idx_flat, dist_flat, W1[2 * CODE_DIM], True)
    h = _post(S, W2, b2.reshape(1, CODE_DIM), h,
              g.reshape(1, CODE_DIM), be.reshape(1, CODE_DIM))

    W1, b1, W2, b2, g, be = params[1]
    A, Bc = _pre(h, W1[:CODE_DIM], W1[CODE_DIM:2 * CODE_DIM],
                 b1.reshape(1, HIDDEN_DIM))
    S = _msg(A, Bc, idx_flat, dist_flat, W1[2 * CODE_DIM], False)
    hg = _post(S[N:], W2, b2.reshape(1, CODE_DIM), h[N:],
               g.reshape(1, CODE_DIM), be.reshape(1, CODE_DIM))
    return hg.reshape(B, N_GRID, CODE_DIM)


# speedup vs baseline: 22.3917x; 1.3177x over previous
import functools

import jax
import jax.numpy as jnp
from jax import lax
from jax.experimental import pallas as pl
from jax.experimental.pallas import tpu as pltpu
from jax.experimental.pallas import tpu_sc as plsc

N_ATOM_TYPES = 16
GRID_SIZE = 8
CODE_DIM = 128
HIDDEN_DIM = 256
K_ATOM = 8
K_GRID = 32
B = 4
N_PER = 1024
N = B * N_PER
N_GRID = GRID_SIZE ** 3
NG = B * N_GRID
NT = N + NG
E_ATOM = N * K_ATOM
E_GRID = NG * K_GRID
E_TOT = E_ATOM + E_GRID

NW = 32


def _knn_body(q_ref, k_ref, a2_ref, b2_ref, idx_ref, dist_ref, *, k, nq, diag):
    b = pl.program_id(0)
    q = q_ref[...]
    kp = k_ref[...]
    g = lax.dot_general(q, kp, (((1,), (1,)), ((), ())),
                        preferred_element_type=jnp.float32)
    d2 = (a2_ref[...] + b2_ref[...]) - 2.0 * g
    jj = lax.broadcasted_iota(jnp.int32, (nq, N_PER), 1)
    if diag:
        ii = lax.broadcasted_iota(jnp.int32, (nq, N_PER), 0)
        d2 = jnp.where(ii == jj, jnp.inf, d2)
    for t in range(k):
        m = jnp.min(d2, axis=1, keepdims=True)
        am = jnp.min(jnp.where(d2 <= m, jj, jnp.int32(2**30)),
                     axis=1, keepdims=True)
        idx_ref[:, t:t + 1] = am + b * N_PER
        dist_ref[:, t:t + 1] = jnp.sqrt(jnp.maximum(m, 0.0))
        d2 = jnp.where(jj == am, jnp.inf, d2)


def _knn_atoms(pos, p2col, p2row):
    return pl.pallas_call(
        functools.partial(_knn_body, k=K_ATOM, nq=N_PER, diag=True),
        grid=(B,),
        in_specs=[pl.BlockSpec((N_PER, 3), lambda b: (b, 0)),
                  pl.BlockSpec((N_PER, 3), lambda b: (b, 0)),
                  pl.BlockSpec((N_PER, 1), lambda b: (b, 0)),
                  pl.BlockSpec((1, N_PER), lambda b: (0, b))],
        out_specs=[pl.BlockSpec((N_PER, K_ATOM), lambda b: (b, 0)),
                   pl.BlockSpec((N_PER, K_ATOM), lambda b: (b, 0))],
        out_shape=[jax.ShapeDtypeStruct((N, K_ATOM), jnp.int32),
                   jax.ShapeDtypeStruct((N, K_ATOM), jnp.float32)],
    )(pos, pos, p2col, p2row)


def _knn_grid(gc, pos, g2col, p2row):
    return pl.pallas_call(
        functools.partial(_knn_body, k=K_GRID, nq=N_GRID, diag=False),
        grid=(B,),
        in_specs=[pl.BlockSpec((N_GRID, 3), lambda b: (0, 0)),
                  pl.BlockSpec((N_PER, 3), lambda b: (b, 0)),
                  pl.BlockSpec((N_GRID, 1), lambda b: (0, 0)),
                  pl.BlockSpec((1, N_PER), lambda b: (0, b))],
        out_specs=[pl.BlockSpec((N_GRID, K_GRID), lambda b: (b, 0)),
                   pl.BlockSpec((N_GRID, K_GRID), lambda b: (b, 0))],
        out_shape=[jax.ShapeDtypeStruct((NG, K_GRID), jnp.int32),
                   jax.ShapeDtypeStruct((NG, K_GRID), jnp.float32)],
    )(gc, pos, g2col, p2row)


def _pre_body(h_ref, wa_ref, wb_ref, b1_ref, a_ref, bc_ref):
    h = h_ref[...]
    a_ref[...] = jnp.dot(h, wa_ref[...], preferred_element_type=jnp.float32)
    bc_ref[...] = (jnp.dot(h, wb_ref[...], preferred_element_type=jnp.float32)
                   + b1_ref[...])


def _pre(h, wa, wb, b1):
    rows = h.shape[0]
    blk = 512
    return pl.pallas_call(
        _pre_body,
        grid=(rows // blk,),
        in_specs=[pl.BlockSpec((blk, CODE_DIM), lambda i: (i, 0)),
                  pl.BlockSpec((CODE_DIM, HIDDEN_DIM), lambda i: (0, 0)),
                  pl.BlockSpec((CODE_DIM, HIDDEN_DIM), lambda i: (0, 0)),
                  pl.BlockSpec((1, HIDDEN_DIM), lambda i: (0, 0))],
        out_specs=[pl.BlockSpec((blk, HIDDEN_DIM), lambda i: (i, 0)),
                   pl.BlockSpec((blk, HIDDEN_DIM), lambda i: (i, 0))],
        out_shape=[jax.ShapeDtypeStruct((rows, HIDDEN_DIM), jnp.float32),
                   jax.ShapeDtypeStruct((rows, HIDDEN_DIM), jnp.float32)],
    )(h, wa, wb, b1)


def _post_body(s_ref, w2_ref, b2_ref, h_ref, g_ref, be_ref, o_ref):
    aggr = (jnp.dot(s_ref[...], w2_ref[...], preferred_element_type=jnp.float32)
            + b2_ref[...])
    hn = h_ref[...] + aggr
    mu = jnp.mean(hn, axis=1, keepdims=True)
    cen = hn - mu
    var = jnp.mean(cen * cen, axis=1, keepdims=True)
    o_ref[...] = cen / jnp.sqrt(var + 1e-5) * g_ref[...] + be_ref[...]


def _post(s, w2, b2, h, g, be):
    rows = h.shape[0]
    blk = 512
    return pl.pallas_call(
        _post_body,
        grid=(rows // blk,),
        in_specs=[pl.BlockSpec((blk, HIDDEN_DIM), lambda i: (i, 0)),
                  pl.BlockSpec((HIDDEN_DIM, CODE_DIM), lambda i: (0, 0)),
                  pl.BlockSpec((1, CODE_DIM), lambda i: (0, 0)),
                  pl.BlockSpec((blk, CODE_DIM), lambda i: (i, 0)),
                  pl.BlockSpec((1, CODE_DIM), lambda i: (0, 0)),
                  pl.BlockSpec((1, CODE_DIM), lambda i: (0, 0))],
        out_specs=pl.BlockSpec((blk, CODE_DIM), lambda i: (i, 0)),
        out_shape=jax.ShapeDtypeStruct((rows, CODE_DIM), jnp.float32),
    )(s, w2, b2, h, g, be)


_EPB_A = 128 // K_ATOM
_EPB_G = 128 // K_GRID
_BLKS_A = N // NW // _EPB_A
_BLKS_G = NG // NW // _EPB_G


_EW_A = N // NW * K_ATOM
_EW_G = NG // NW * K_GRID
_CW_A = N // NW
_CW_G = NG // NW


def _sc_msg_body(include_atoms, a_hbm, bc_hbm, idx_hbm, dist_hbm, w_hbm,
                 s_hbm, w_v, idx_all, dist_all, bc_all, s_v,
                 rows_a, rows_b, sem_a, sem_b):
    cid = lax.axis_index("c")
    sid = lax.axis_index("s")
    wid = sid * 2 + cid

    pltpu.sync_copy(w_hbm, w_v)
    if include_atoms:
        pltpu.sync_copy(idx_hbm.at[pl.ds(wid * _EW_A, _EW_A)],
                        idx_all.at[pl.ds(0, _EW_A)])
        pltpu.sync_copy(dist_hbm.at[pl.ds(wid * _EW_A, _EW_A)],
                        dist_all.at[pl.ds(0, _EW_A)])
        pltpu.sync_copy(bc_hbm.at[pl.ds(wid * _CW_A, _CW_A)],
                        bc_all.at[pl.ds(0, _CW_A)])
    pltpu.sync_copy(idx_hbm.at[pl.ds(E_ATOM + wid * _EW_G, _EW_G)],
                    idx_all.at[pl.ds(_EW_A, _EW_G)])
    pltpu.sync_copy(dist_hbm.at[pl.ds(E_ATOM + wid * _EW_G, _EW_G)],
                    dist_all.at[pl.ds(_EW_A, _EW_G)])
    pltpu.sync_copy(bc_hbm.at[pl.ds(N + wid * _CW_G, _CW_G)],
                    bc_all.at[pl.ds(_CW_A, _CW_G)])
    ws = [w_v[pl.ds(i * 16, 16)] for i in range(16)]

    def issue(rows, sem, e_loc):
        pltpu.async_copy(a_hbm.at[idx_all.at[pl.ds(e_loc, 128)]], rows, sem)

    def wait(rows, sem):
        pltpu.make_async_copy(a_hbm.at[idx_all.at[pl.ds(0, 128)]],
                              rows, sem).wait()

    def compute(rows, e_loc, cb, c_hbm0, nc, k, unroll):
        inv = jnp.float32(1.0 / k)

        def center(ci, _):
            cl = cb + ci
            for half in range(2):
                bcs = [bc_all[cl, pl.ds((half * 8 + s) * 16, 16)]
                       for s in range(8)]

                def edges(j0, acc):
                    for ju in range(unroll):
                        r = ci * k + j0 + ju
                        dvec = jnp.full(
                            (16,),
                            dist_all[pl.ds(e_loc + r, 16)][0], jnp.float32)
                        for s in range(8):
                            t = (rows[r, pl.ds((half * 8 + s) * 16, 16)]
                                 + bcs[s] + dvec * ws[half * 8 + s])
                            acc[s] = acc[s] + jnp.maximum(t, jnp.float32(0.0))
                    return acc

                acc = [jnp.zeros((16,), jnp.float32) for _ in range(8)]
                if k <= 8:
                    acc = edges(0, acc)
                else:
                    acc = lax.fori_loop(
                        0, k // unroll,
                        lambda p, a: edges(p * unroll, a), acc)
                for s in range(8):
                    s_v[ci, pl.ds((half * 8 + s) * 16, 16)] = acc[s] * inv
            return _

        lax.fori_loop(0, nc, center, None)
        pltpu.sync_copy(s_v.at[pl.ds(0, nc)], s_hbm.at[pl.ds(c_hbm0, nc)])

    def phase(e_loc0, cb0, c_hbm0, nblk, nc, k, unroll):
        issue(rows_a, sem_a, e_loc0)

        def pair(p, _):
            e_cur = e_loc0 + p * 256
            issue(rows_b, sem_b, e_cur + 128)
            wait(rows_a, sem_a)
            compute(rows_a, e_cur, cb0 + 2 * p * nc, c_hbm0 + 2 * p * nc,
                    nc, k, unroll)

            @pl.when(p + 1 < nblk // 2)
            def _issue_next():
                issue(rows_a, sem_a, e_cur + 256)

            wait(rows_b, sem_b)
            compute(rows_b, e_cur + 128, cb0 + (2 * p + 1) * nc,
                    c_hbm0 + (2 * p + 1) * nc, nc, k, unroll)
            return _

        lax.fori_loop(0, nblk // 2, pair, None)

    if include_atoms:
        phase(0, 0, wid * _CW_A, _BLKS_A, _EPB_A, K_ATOM, K_ATOM)
    phase(_EW_A, _CW_A, N + wid * _CW_G, _BLKS_G, _EPB_G, K_GRID, 4)


def _msg(a, bc, idx, dist, w, include_atoms):
    mesh = plsc.VectorSubcoreMesh(core_axis_name="c", subcore_axis_name="s")
    f = pl.kernel(
        functools.partial(_sc_msg_body, include_atoms),
        out_type=jax.ShapeDtypeStruct((NT, HIDDEN_DIM), jnp.float32),
        mesh=mesh,
        scratch_types=[
            pltpu.VMEM((HIDDEN_DIM,), jnp.float32),
            pltpu.VMEM((_EW_A + _EW_G,), jnp.int32),
            pltpu.VMEM((_EW_A + _EW_G + 16,), jnp.float32),
            pltpu.VMEM((_CW_A + _CW_G, HIDDEN_DIM), jnp.float32),
            pltpu.VMEM((_EPB_A, HIDDEN_DIM), jnp.float32),
            pltpu.VMEM((128, HIDDEN_DIM), jnp.float32),
            pltpu.VMEM((128, HIDDEN_DIM), jnp.float32),
            pltpu.SemaphoreType.DMA,
            pltpu.SemaphoreType.DMA,
        ],
    )
    return f(a, bc, idx, dist, w)


def kernel(pos, x, batch, W1_0, b1_0, W2_0, b2_0, g_0, be_0,
           W1_1, b1_1, W2_1, b2_1, g_1, be_1):
    f32 = jnp.float32
    pos = pos.astype(f32)

    p2 = jnp.sum(pos * pos, axis=1)
    p2col = p2.reshape(N, 1)
    p2row = p2.reshape(1, N)

    g1d = jnp.linspace(-1.0, 1.0, GRID_SIZE)
    mx, my, mz = jnp.meshgrid(g1d, g1d, g1d, indexing='ij')
    gc = jnp.stack([mx, my, mz], axis=-1).reshape(-1, 3).astype(f32)
    g2col = jnp.sum(gc * gc, axis=1).reshape(N_GRID, 1)

    idxA, distA = _knn_atoms(pos, p2col, p2row)
    idxG, distG = _knn_grid(gc, pos, g2col, p2row)
    idx_flat = jnp.concatenate([idxA.reshape(-1), idxG.reshape(-1)])
    dist_flat = jnp.concatenate([distA.reshape(-1), distG.reshape(-1)])

    h0a = jax.nn.one_hot(x, N_ATOM_TYPES, dtype=f32)
    h = jnp.concatenate(
        [jnp.pad(h0a, ((0, 0), (0, CODE_DIM - N_ATOM_TYPES))),
         jnp.zeros((NG, CODE_DIM), f32)], axis=0)

    params = [(W1_0, b1_0, W2_0, b2_0, g_0, be_0),
              (W1_1, b1_1, W2_1, b2_1, g_1, be_1)]

    W1, b1, W2, b2, g, be = params[0]
    A, Bc = _pre(h, W1[:CODE_DIM], W1[CODE_DIM:2 * CODE_DIM],
                 b1.reshape(1, HIDDEN_DIM))
    S = _msg(A, Bc, idx_flat, dist_flat, W1[2 * CODE_DIM], True)
    h = _post(S, W2, b2.reshape(1, CODE_DIM), h,
              g.reshape(1, CODE_DIM), be.reshape(1, CODE_DIM))

    W1, b1, W2, b2, g, be = params[1]
    A, Bc = _pre(h, W1[:CODE_DIM], W1[CODE_DIM:2 * CODE_DIM],
                 b1.reshape(1, HIDDEN_DIM))
    S = _msg(A, Bc, idx_flat, dist_flat, W1[2 * CODE_DIM], False)
    hg = _post(S[N:], W2, b2.reshape(1, CODE_DIM), h[N:],
               g.reshape(1, CODE_DIM), be.reshape(1, CODE_DIM))
    return hg.reshape(B, N_GRID, CODE_DIM)


# speedup vs baseline: 23.2985x; 1.0405x over previous
import functools

import jax
import jax.numpy as jnp
from jax import lax
from jax.experimental import pallas as pl
from jax.experimental.pallas import tpu as pltpu
from jax.experimental.pallas import tpu_sc as plsc

N_ATOM_TYPES = 16
GRID_SIZE = 8
CODE_DIM = 128
HIDDEN_DIM = 256
K_ATOM = 8
K_GRID = 32
B = 4
N_PER = 1024
N = B * N_PER
N_GRID = GRID_SIZE ** 3
NG = B * N_GRID
NT = N + NG
E_ATOM = N * K_ATOM
E_GRID = NG * K_GRID
E_TOT = E_ATOM + E_GRID

NW = 32


def _topk_store(d2, k, b, idx_ref, dist_ref):
    nq = d2.shape[0]
    jj = lax.broadcasted_iota(jnp.int32, (nq, N_PER), 1)
    for t in range(k):
        m = jnp.min(d2, axis=1, keepdims=True)
        am = jnp.min(jnp.where(d2 <= m, jj, jnp.int32(2**30)),
                     axis=1, keepdims=True)
        idx_ref[:, t:t + 1] = am + b * N_PER
        dist_ref[:, t:t + 1] = jnp.sqrt(jnp.maximum(m, 0.0))
        d2 = jnp.where(jj == am, jnp.inf, d2)


def _knn_body(pos_ref, gc_ref, p2c_ref, p2r_ref, g2c_ref,
              idxa_ref, dista_ref, idxg_ref, distg_ref):
    b = pl.program_id(0)
    kp = pos_ref[...]
    p2r = p2r_ref[...]
    g = lax.dot_general(kp, kp, (((1,), (1,)), ((), ())),
                        preferred_element_type=jnp.float32)
    d2 = (p2c_ref[...] + p2r) - 2.0 * g
    ii = lax.broadcasted_iota(jnp.int32, (N_PER, N_PER), 0)
    jj = lax.broadcasted_iota(jnp.int32, (N_PER, N_PER), 1)
    d2 = jnp.where(ii == jj, jnp.inf, d2)
    _topk_store(d2, K_ATOM, b, idxa_ref, dista_ref)

    gq = gc_ref[...]
    gg = lax.dot_general(gq, kp, (((1,), (1,)), ((), ())),
                         preferred_element_type=jnp.float32)
    d2g = (g2c_ref[...] + p2r) - 2.0 * gg
    _topk_store(d2g, K_GRID, b, idxg_ref, distg_ref)


def _knn(pos, gc, p2col, p2row, g2col):
    return pl.pallas_call(
        _knn_body,
        grid=(B,),
        in_specs=[pl.BlockSpec((N_PER, 3), lambda b: (b, 0)),
                  pl.BlockSpec((N_GRID, 3), lambda b: (0, 0)),
                  pl.BlockSpec((N_PER, 1), lambda b: (b, 0)),
                  pl.BlockSpec((1, N_PER), lambda b: (0, b)),
                  pl.BlockSpec((N_GRID, 1), lambda b: (0, 0))],
        out_specs=[pl.BlockSpec((N_PER, K_ATOM), lambda b: (b, 0)),
                   pl.BlockSpec((N_PER, K_ATOM), lambda b: (b, 0)),
                   pl.BlockSpec((N_GRID, K_GRID), lambda b: (b, 0)),
                   pl.BlockSpec((N_GRID, K_GRID), lambda b: (b, 0))],
        out_shape=[jax.ShapeDtypeStruct((N, K_ATOM), jnp.int32),
                   jax.ShapeDtypeStruct((N, K_ATOM), jnp.float32),
                   jax.ShapeDtypeStruct((NG, K_GRID), jnp.int32),
                   jax.ShapeDtypeStruct((NG, K_GRID), jnp.float32)],
    )(pos, gc, p2col, p2row, g2col)


_BLK = 512
_NBLK_AT = N // _BLK
_NBLK = NT // _BLK


def _onehot128(x_col):
    jj = lax.broadcasted_iota(jnp.int32, (x_col.shape[0], CODE_DIM), 1)
    return jnp.where(jj == x_col, jnp.float32(1.0), jnp.float32(0.0))


def _pre0_body(x_ref, wa_ref, wb_ref, b1_ref, a_ref, bc_ref):
    oh = _onehot128(x_ref[...])
    a_ref[...] = jnp.dot(oh, wa_ref[...], preferred_element_type=jnp.float32)
    bc_ref[...] = (jnp.dot(oh, wb_ref[...],
                           preferred_element_type=jnp.float32) + b1_ref[...])


def _pre0(x_col, wa, wb, b1):
    return pl.pallas_call(
        _pre0_body,
        grid=(_NBLK_AT,),
        in_specs=[pl.BlockSpec((_BLK, 1), lambda i: (i, 0)),
                  pl.BlockSpec((CODE_DIM, HIDDEN_DIM), lambda i: (0, 0)),
                  pl.BlockSpec((CODE_DIM, HIDDEN_DIM), lambda i: (0, 0)),
                  pl.BlockSpec((1, HIDDEN_DIM), lambda i: (0, 0))],
        out_specs=[pl.BlockSpec((_BLK, HIDDEN_DIM), lambda i: (i, 0)),
                   pl.BlockSpec((_BLK, HIDDEN_DIM), lambda i: (i, 0))],
        out_shape=[jax.ShapeDtypeStruct((N, HIDDEN_DIM), jnp.float32),
                   jax.ShapeDtypeStruct((N, HIDDEN_DIM), jnp.float32)],
    )(x_col, wa, wb, b1)


def _ln(hn, g, be):
    mu = jnp.mean(hn, axis=1, keepdims=True)
    cen = hn - mu
    var = jnp.mean(cen * cen, axis=1, keepdims=True)
    return cen / jnp.sqrt(var + 1e-5) * g + be


def _fuse_body(s_ref, w2_ref, b2_ref, x_ref, g_ref, be_ref,
               wa_ref, wb_ref, b1_ref, h_ref, a_ref, bc_ref):
    i = pl.program_id(0)
    aggr = (jnp.dot(s_ref[...], w2_ref[...],
                    preferred_element_type=jnp.float32) + b2_ref[...])
    h0 = jnp.where(i < _NBLK_AT, _onehot128(x_ref[...]), jnp.float32(0.0))
    h1 = _ln(h0 + aggr, g_ref[...], be_ref[...])
    h_ref[...] = h1
    a_ref[...] = jnp.dot(h1, wa_ref[...], preferred_element_type=jnp.float32)
    bc_ref[...] = (jnp.dot(h1, wb_ref[...],
                           preferred_element_type=jnp.float32) + b1_ref[...])


def _fuse(s, w2, b2, x_col, g, be, wa, wb, b1):
    return pl.pallas_call(
        _fuse_body,
        grid=(_NBLK,),
        in_specs=[pl.BlockSpec((_BLK, HIDDEN_DIM), lambda i: (i, 0)),
                  pl.BlockSpec((HIDDEN_DIM, CODE_DIM), lambda i: (0, 0)),
                  pl.BlockSpec((1, CODE_DIM), lambda i: (0, 0)),
                  pl.BlockSpec((_BLK, 1),
                               lambda i: (jnp.minimum(i, _NBLK_AT - 1), 0)),
                  pl.BlockSpec((1, CODE_DIM), lambda i: (0, 0)),
                  pl.BlockSpec((1, CODE_DIM), lambda i: (0, 0)),
                  pl.BlockSpec((CODE_DIM, HIDDEN_DIM), lambda i: (0, 0)),
                  pl.BlockSpec((CODE_DIM, HIDDEN_DIM), lambda i: (0, 0)),
                  pl.BlockSpec((1, HIDDEN_DIM), lambda i: (0, 0))],
        out_specs=[pl.BlockSpec((_BLK, CODE_DIM), lambda i: (i, 0)),
                   pl.BlockSpec((_BLK, HIDDEN_DIM), lambda i: (i, 0)),
                   pl.BlockSpec((_BLK, HIDDEN_DIM), lambda i: (i, 0))],
        out_shape=[jax.ShapeDtypeStruct((NT, CODE_DIM), jnp.float32),
                   jax.ShapeDtypeStruct((NT, HIDDEN_DIM), jnp.float32),
                   jax.ShapeDtypeStruct((NT, HIDDEN_DIM), jnp.float32)],
    )(s, w2, b2, x_col, g, be, wa, wb, b1)


def _post1_body(s_ref, w2_ref, b2_ref, h_ref, g_ref, be_ref, o_ref):
    aggr = (jnp.dot(s_ref[...], w2_ref[...],
                    preferred_element_type=jnp.float32) + b2_ref[...])
    o_ref[...] = _ln(h_ref[...] + aggr, g_ref[...], be_ref[...])


def _post1(s, w2, b2, h, g, be):
    return pl.pallas_call(
        _post1_body,
        grid=(NG // _BLK,),
        in_specs=[pl.BlockSpec((_BLK, HIDDEN_DIM),
                               lambda i: (i + _NBLK_AT, 0)),
                  pl.BlockSpec((HIDDEN_DIM, CODE_DIM), lambda i: (0, 0)),
                  pl.BlockSpec((1, CODE_DIM), lambda i: (0, 0)),
                  pl.BlockSpec((_BLK, CODE_DIM),
                               lambda i: (i + _NBLK_AT, 0)),
                  pl.BlockSpec((1, CODE_DIM), lambda i: (0, 0)),
                  pl.BlockSpec((1, CODE_DIM), lambda i: (0, 0))],
        out_specs=pl.BlockSpec((_BLK, CODE_DIM), lambda i: (i, 0)),
        out_shape=jax.ShapeDtypeStruct((NG, CODE_DIM), jnp.float32),
    )(s, w2, b2, h, g, be)


_EPB_A = 128 // K_ATOM
_EPB_G = 128 // K_GRID
_BLKS_A = N // NW // _EPB_A
_BLKS_G = NG // NW // _EPB_G


_EW_A = N // NW * K_ATOM
_EW_G = NG // NW * K_GRID
_CW_A = N // NW
_CW_G = NG // NW


def _sc_msg_body(include_atoms, a_hbm, bc_hbm, idx_hbm, dist_hbm, w_hbm,
                 s_hbm, w_v, idx_all, dist_all, bc_all, s_v,
                 rows_a, rows_b, sem_a, sem_b):
    cid = lax.axis_index("c")
    sid = lax.axis_index("s")
    wid = sid * 2 + cid

    pltpu.sync_copy(w_hbm, w_v)
    if include_atoms:
        pltpu.sync_copy(idx_hbm.at[pl.ds(wid * _EW_A, _EW_A)],
                        idx_all.at[pl.ds(0, _EW_A)])
        pltpu.sync_copy(dist_hbm.at[pl.ds(wid * _EW_A, _EW_A)],
                        dist_all.at[pl.ds(0, _EW_A)])
        pltpu.sync_copy(bc_hbm.at[pl.ds(wid * _CW_A, _CW_A)],
                        bc_all.at[pl.ds(0, _CW_A)])
    pltpu.sync_copy(idx_hbm.at[pl.ds(E_ATOM + wid * _EW_G, _EW_G)],
                    idx_all.at[pl.ds(_EW_A, _EW_G)])
    pltpu.sync_copy(dist_hbm.at[pl.ds(E_ATOM + wid * _EW_G, _EW_G)],
                    dist_all.at[pl.ds(_EW_A, _EW_G)])
    pltpu.sync_copy(bc_hbm.at[pl.ds(N + wid * _CW_G, _CW_G)],
                    bc_all.at[pl.ds(_CW_A, _CW_G)])
    ws = [w_v[pl.ds(i * 16, 16)] for i in range(16)]

    def issue(rows, sem, e_loc):
        pltpu.async_copy(a_hbm.at[idx_all.at[pl.ds(e_loc, 128)]], rows, sem)

    def wait(rows, sem):
        pltpu.make_async_copy(a_hbm.at[idx_all.at[pl.ds(0, 128)]],
                              rows, sem).wait()

    def compute(rows, e_loc, cb, c_hbm0, nc, k, unroll):
        inv = jnp.float32(1.0 / k)

        def center(ci, _):
            cl = cb + ci
            for half in range(2):
                bcs = [bc_all[cl, pl.ds((half * 8 + s) * 16, 16)]
                       for s in range(8)]

                def edges(j0, acc):
                    for ju in range(unroll):
                        r = ci * k + j0 + ju
                        dvec = jnp.full(
                            (16,),
                            dist_all[pl.ds(e_loc + r, 16)][0], jnp.float32)
                        for s in range(8):
                            t = (rows[r, pl.ds((half * 8 + s) * 16, 16)]
                                 + bcs[s] + dvec * ws[half * 8 + s])
                            acc[s] = acc[s] + jnp.maximum(t, jnp.float32(0.0))
                    return acc

                acc = [jnp.zeros((16,), jnp.float32) for _ in range(8)]
                if k <= 8:
                    acc = edges(0, acc)
                else:
                    acc = lax.fori_loop(
                        0, k // unroll,
                        lambda p, a: edges(p * unroll, a), acc)
                for s in range(8):
                    s_v[ci, pl.ds((half * 8 + s) * 16, 16)] = acc[s] * inv
            return _

        lax.fori_loop(0, nc, center, None)
        pltpu.sync_copy(s_v.at[pl.ds(0, nc)], s_hbm.at[pl.ds(c_hbm0, nc)])

    def phase(e_loc0, cb0, c_hbm0, nblk, nc, k, unroll):
        issue(rows_a, sem_a, e_loc0)

        def pair(p, _):
            e_cur = e_loc0 + p * 256
            issue(rows_b, sem_b, e_cur + 128)
            wait(rows_a, sem_a)
            compute(rows_a, e_cur, cb0 + 2 * p * nc, c_hbm0 + 2 * p * nc,
                    nc, k, unroll)

            @pl.when(p + 1 < nblk // 2)
            def _issue_next():
                issue(rows_a, sem_a, e_cur + 256)

            wait(rows_b, sem_b)
            compute(rows_b, e_cur + 128, cb0 + (2 * p + 1) * nc,
                    c_hbm0 + (2 * p + 1) * nc, nc, k, unroll)
            return _

        lax.fori_loop(0, nblk // 2, pair, None)

    if include_atoms:
        phase(0, 0, wid * _CW_A, _BLKS_A, _EPB_A, K_ATOM, K_ATOM)
    phase(_EW_A, _CW_A, N + wid * _CW_G, _BLKS_G, _EPB_G, K_GRID, 4)


def _msg(a, bc, idx, dist, w, include_atoms):
    mesh = plsc.VectorSubcoreMesh(core_axis_name="c", subcore_axis_name="s")
    f = pl.kernel(
        functools.partial(_sc_msg_body, include_atoms),
        out_type=jax.ShapeDtypeStruct((NT, HIDDEN_DIM), jnp.float32),
        mesh=mesh,
        scratch_types=[
            pltpu.VMEM((HIDDEN_DIM,), jnp.float32),
            pltpu.VMEM((_EW_A + _EW_G,), jnp.int32),
            pltpu.VMEM((_EW_A + _EW_G + 16,), jnp.float32),
            pltpu.VMEM((_CW_A + _CW_G, HIDDEN_DIM), jnp.float32),
            pltpu.VMEM((_EPB_A, HIDDEN_DIM), jnp.float32),
            pltpu.VMEM((128, HIDDEN_DIM), jnp.float32),
            pltpu.VMEM((128, HIDDEN_DIM), jnp.float32),
            pltpu.SemaphoreType.DMA,
            pltpu.SemaphoreType.DMA,
        ],
    )
    return f(a, bc, idx, dist, w)


def kernel(pos, x, batch, W1_0, b1_0, W2_0, b2_0, g_0, be_0,
           W1_1, b1_1, W2_1, b2_1, g_1, be_1):
    f32 = jnp.float32
    pos = pos.astype(f32)

    p2 = jnp.sum(pos * pos, axis=1)
    p2col = p2.reshape(N, 1)
    p2row = p2.reshape(1, N)

    g1d = jnp.linspace(-1.0, 1.0, GRID_SIZE)
    mx, my, mz = jnp.meshgrid(g1d, g1d, g1d, indexing='ij')
    gc = jnp.stack([mx, my, mz], axis=-1).reshape(-1, 3).astype(f32)
    g2col = jnp.sum(gc * gc, axis=1).reshape(N_GRID, 1)

    idxA, distA, idxG, distG = _knn(pos, gc, p2col, p2row, g2col)
    idx_flat = jnp.concatenate([idxA.reshape(-1), idxG.reshape(-1)])
    dist_flat = jnp.concatenate([distA.reshape(-1), distG.reshape(-1)])

    x_col = x.astype(jnp.int32).reshape(N, 1)

    b1r_0 = b1_0.reshape(1, HIDDEN_DIM)
    A0, Bc0_at = _pre0(x_col, W1_0[:CODE_DIM], W1_0[CODE_DIM:2 * CODE_DIM],
                       b1r_0)
    Bc0 = jnp.concatenate(
        [Bc0_at, jnp.broadcast_to(b1r_0, (NG, HIDDEN_DIM))], axis=0)
    S0 = _msg(A0, Bc0, idx_flat, dist_flat, W1_0[2 * CODE_DIM], True)

    h1, A1, Bc1 = _fuse(S0, W2_0, b2_0.reshape(1, CODE_DIM), x_col,
                        g_0.reshape(1, CODE_DIM), be_0.reshape(1, CODE_DIM),
                        W1_1[:CODE_DIM], W1_1[CODE_DIM:2 * CODE_DIM],
                        b1_1.reshape(1, HIDDEN_DIM))

    S1 = _msg(A1, Bc1, idx_flat, dist_flat, W1_1[2 * CODE_DIM], False)
    hg = _post1(S1, W2_1, b2_1.reshape(1, CODE_DIM), h1,
                g_1.reshape(1, CODE_DIM), be_1.reshape(1, CODE_DIM))
    return hg.reshape(B, N_GRID, CODE_DIM)


# speedup vs baseline: 26.2110x; 1.1250x over previous
import functools

import jax
import jax.numpy as jnp
from jax import lax
from jax.experimental import pallas as pl
from jax.experimental.pallas import tpu as pltpu
from jax.experimental.pallas import tpu_sc as plsc

N_ATOM_TYPES = 16
GRID_SIZE = 8
CODE_DIM = 128
HIDDEN_DIM = 256
K_ATOM = 8
K_GRID = 32
B = 4
N_PER = 1024
N = B * N_PER
N_GRID = GRID_SIZE ** 3
NG = B * N_GRID
NT = N + NG
E_ATOM = N * K_ATOM
E_GRID = NG * K_GRID
E_TOT = E_ATOM + E_GRID

NW = 32


def _topk_store(d2, k, b, idx_ref, dist_ref):
    nq = d2.shape[0]
    jj = lax.broadcasted_iota(jnp.int32, (nq, N_PER), 1)
    for t in range(k):
        m = jnp.min(d2, axis=1, keepdims=True)
        am = jnp.min(jnp.where(d2 <= m, jj, jnp.int32(2**30)),
                     axis=1, keepdims=True)
        idx_ref[:, t:t + 1] = am + b * N_PER
        dist_ref[:, t:t + 1] = jnp.sqrt(jnp.maximum(m, 0.0))
        d2 = jnp.where(jj == am, jnp.inf, d2)


def _knn_atoms_body(pos_ref, p2c_ref, p2r_ref, idxa_ref, dista_ref):
    b = pl.program_id(0)
    kp = pos_ref[...]
    g = lax.dot_general(kp, kp, (((1,), (1,)), ((), ())),
                        preferred_element_type=jnp.float32)
    d2 = (p2c_ref[...] + p2r_ref[...]) - 2.0 * g
    ii = lax.broadcasted_iota(jnp.int32, (N_PER, N_PER), 0)
    jj = lax.broadcasted_iota(jnp.int32, (N_PER, N_PER), 1)
    d2 = jnp.where(ii == jj, jnp.inf, d2)
    _topk_store(d2, K_ATOM, b, idxa_ref, dista_ref)


def _knn_grid_body(gc_ref, pos_ref, g2c_ref, p2r_ref, idxg_ref, distg_ref):
    b = pl.program_id(0)
    kp = pos_ref[...]
    gq = gc_ref[...]
    gg = lax.dot_general(gq, kp, (((1,), (1,)), ((), ())),
                         preferred_element_type=jnp.float32)
    d2g = (g2c_ref[...] + p2r_ref[...]) - 2.0 * gg
    _topk_store(d2g, K_GRID, b, idxg_ref, distg_ref)


def _knn_atoms(pos, p2col, p2row):
    return pl.pallas_call(
        _knn_atoms_body,
        grid=(B,),
        in_specs=[pl.BlockSpec((N_PER, 3), lambda b: (b, 0)),
                  pl.BlockSpec((N_PER, 1), lambda b: (b, 0)),
                  pl.BlockSpec((1, N_PER), lambda b: (0, b))],
        out_specs=[pl.BlockSpec((N_PER, K_ATOM), lambda b: (b, 0)),
                   pl.BlockSpec((N_PER, K_ATOM), lambda b: (b, 0))],
        out_shape=[jax.ShapeDtypeStruct((N, K_ATOM), jnp.int32),
                   jax.ShapeDtypeStruct((N, K_ATOM), jnp.float32)],
    )(pos, p2col, p2row)


def _knn_grid(gc, pos, g2col, p2row):
    return pl.pallas_call(
        _knn_grid_body,
        grid=(B,),
        in_specs=[pl.BlockSpec((N_GRID, 3), lambda b: (0, 0)),
                  pl.BlockSpec((N_PER, 3), lambda b: (b, 0)),
                  pl.BlockSpec((N_GRID, 1), lambda b: (0, 0)),
                  pl.BlockSpec((1, N_PER), lambda b: (0, b))],
        out_specs=[pl.BlockSpec((N_GRID, K_GRID), lambda b: (b, 0)),
                   pl.BlockSpec((N_GRID, K_GRID), lambda b: (b, 0))],
        out_shape=[jax.ShapeDtypeStruct((NG, K_GRID), jnp.int32),
                   jax.ShapeDtypeStruct((NG, K_GRID), jnp.float32)],
    )(gc, pos, g2col, p2row)


_BLK = 512
_NBLK_AT = N // _BLK
_NBLK = NT // _BLK


def _onehot128(x_col):
    jj = lax.broadcasted_iota(jnp.int32, (x_col.shape[0], CODE_DIM), 1)
    return jnp.where(jj == x_col, jnp.float32(1.0), jnp.float32(0.0))


def _pre0_body(x_ref, wa_ref, wb_ref, b1_ref, a_ref, bc_ref):
    oh = _onehot128(x_ref[...])
    a_ref[...] = jnp.dot(oh, wa_ref[...], preferred_element_type=jnp.float32)
    bc_ref[...] = (jnp.dot(oh, wb_ref[...],
                           preferred_element_type=jnp.float32) + b1_ref[...])


def _pre0(x_col, wa, wb, b1):
    return pl.pallas_call(
        _pre0_body,
        grid=(_NBLK_AT,),
        in_specs=[pl.BlockSpec((_BLK, 1), lambda i: (i, 0)),
                  pl.BlockSpec((CODE_DIM, HIDDEN_DIM), lambda i: (0, 0)),
                  pl.BlockSpec((CODE_DIM, HIDDEN_DIM), lambda i: (0, 0)),
                  pl.BlockSpec((1, HIDDEN_DIM), lambda i: (0, 0))],
        out_specs=[pl.BlockSpec((_BLK, HIDDEN_DIM), lambda i: (i, 0)),
                   pl.BlockSpec((_BLK, HIDDEN_DIM), lambda i: (i, 0))],
        out_shape=[jax.ShapeDtypeStruct((N, HIDDEN_DIM), jnp.float32),
                   jax.ShapeDtypeStruct((N, HIDDEN_DIM), jnp.float32)],
    )(x_col, wa, wb, b1)


def _ln(hn, g, be):
    mu = jnp.mean(hn, axis=1, keepdims=True)
    cen = hn - mu
    var = jnp.mean(cen * cen, axis=1, keepdims=True)
    return cen / jnp.sqrt(var + 1e-5) * g + be


def _fuse_body(sa_ref, sg_ref, w2_ref, b2_ref, x_ref, g_ref, be_ref,
               wa_ref, wb_ref, b1_ref, h_ref, a_ref, bc_ref):
    i = pl.program_id(0)
    at = i < _NBLK_AT
    s = jnp.where(at, sa_ref[...], sg_ref[...])
    aggr = (jnp.dot(s, w2_ref[...],
                    preferred_element_type=jnp.float32) + b2_ref[...])
    h0 = jnp.where(at, _onehot128(x_ref[...]), jnp.float32(0.0))
    h1 = _ln(h0 + aggr, g_ref[...], be_ref[...])
    h_ref[...] = h1
    a_ref[...] = jnp.dot(h1, wa_ref[...], preferred_element_type=jnp.float32)
    bc_ref[...] = (jnp.dot(h1, wb_ref[...],
                           preferred_element_type=jnp.float32) + b1_ref[...])


def _fuse(s_at, s_gr, w2, b2, x_col, g, be, wa, wb, b1):
    return pl.pallas_call(
        _fuse_body,
        grid=(_NBLK,),
        in_specs=[pl.BlockSpec((_BLK, HIDDEN_DIM),
                               lambda i: (jnp.minimum(i, _NBLK_AT - 1), 0)),
                  pl.BlockSpec((_BLK, HIDDEN_DIM),
                               lambda i: (jnp.maximum(i - _NBLK_AT, 0), 0)),
                  pl.BlockSpec((HIDDEN_DIM, CODE_DIM), lambda i: (0, 0)),
                  pl.BlockSpec((1, CODE_DIM), lambda i: (0, 0)),
                  pl.BlockSpec((_BLK, 1),
                               lambda i: (jnp.minimum(i, _NBLK_AT - 1), 0)),
                  pl.BlockSpec((1, CODE_DIM), lambda i: (0, 0)),
                  pl.BlockSpec((1, CODE_DIM), lambda i: (0, 0)),
                  pl.BlockSpec((CODE_DIM, HIDDEN_DIM), lambda i: (0, 0)),
                  pl.BlockSpec((CODE_DIM, HIDDEN_DIM), lambda i: (0, 0)),
                  pl.BlockSpec((1, HIDDEN_DIM), lambda i: (0, 0))],
        out_specs=[pl.BlockSpec((_BLK, CODE_DIM), lambda i: (i, 0)),
                   pl.BlockSpec((_BLK, HIDDEN_DIM), lambda i: (i, 0)),
                   pl.BlockSpec((_BLK, HIDDEN_DIM), lambda i: (i, 0))],
        out_shape=[jax.ShapeDtypeStruct((NT, CODE_DIM), jnp.float32),
                   jax.ShapeDtypeStruct((NT, HIDDEN_DIM), jnp.float32),
                   jax.ShapeDtypeStruct((NT, HIDDEN_DIM), jnp.float32)],
    )(s_at, s_gr, w2, b2, x_col, g, be, wa, wb, b1)


def _post1_body(s_ref, w2_ref, b2_ref, h_ref, g_ref, be_ref, o_ref):
    aggr = (jnp.dot(s_ref[...], w2_ref[...],
                    preferred_element_type=jnp.float32) + b2_ref[...])
    o_ref[...] = _ln(h_ref[...] + aggr, g_ref[...], be_ref[...])


def _post1(s, w2, b2, h, g, be):
    return pl.pallas_call(
        _post1_body,
        grid=(NG // _BLK,),
        in_specs=[pl.BlockSpec((_BLK, HIDDEN_DIM), lambda i: (i, 0)),
                  pl.BlockSpec((HIDDEN_DIM, CODE_DIM), lambda i: (0, 0)),
                  pl.BlockSpec((1, CODE_DIM), lambda i: (0, 0)),
                  pl.BlockSpec((_BLK, CODE_DIM),
                               lambda i: (i + _NBLK_AT, 0)),
                  pl.BlockSpec((1, CODE_DIM), lambda i: (0, 0)),
                  pl.BlockSpec((1, CODE_DIM), lambda i: (0, 0))],
        out_specs=pl.BlockSpec((_BLK, CODE_DIM), lambda i: (i, 0)),
        out_shape=jax.ShapeDtypeStruct((NG, CODE_DIM), jnp.float32),
    )(s, w2, b2, h, g, be)


_EPB_A = 128 // K_ATOM
_EPB_G = 128 // K_GRID
_BLKS_A = N // NW // _EPB_A
_BLKS_G = NG // NW // _EPB_G


_EW_A = N // NW * K_ATOM
_EW_G = NG // NW * K_GRID
_CW_A = N // NW
_CW_G = NG // NW


def _sc_msg_body(is_grid, a_hbm, bc_hbm, idx_hbm, dist_hbm, w_hbm,
                 s_hbm, w_v, idx_all, dist_all, bc_all, s_v,
                 rows_a, rows_b, sem_a, sem_b):
    cid = lax.axis_index("c")
    sid = lax.axis_index("s")
    wid = sid * 2 + cid
    if is_grid:
        ew, cw, k, nblk, nc, unroll = _EW_G, _CW_G, K_GRID, _BLKS_G, _EPB_G, 4
        bc0 = N + wid * cw
    else:
        ew, cw, k, nblk, nc, unroll = (_EW_A, _CW_A, K_ATOM, _BLKS_A,
                                       _EPB_A, K_ATOM)
        bc0 = wid * cw

    pltpu.sync_copy(w_hbm, w_v)
    pltpu.sync_copy(idx_hbm.at[pl.ds(wid * ew, ew)], idx_all.at[pl.ds(0, ew)])
    pltpu.sync_copy(dist_hbm.at[pl.ds(wid * ew, ew)],
                    dist_all.at[pl.ds(0, ew)])
    pltpu.sync_copy(bc_hbm.at[pl.ds(bc0, cw)], bc_all.at[pl.ds(0, cw)])
    ws = [w_v[pl.ds(i * 16, 16)] for i in range(16)]

    def issue(rows, sem, e_loc):
        pltpu.async_copy(a_hbm.at[idx_all.at[pl.ds(e_loc, 128)]], rows, sem)

    def wait(rows, sem):
        pltpu.make_async_copy(a_hbm.at[idx_all.at[pl.ds(0, 128)]],
                              rows, sem).wait()

    inv = jnp.float32(1.0 / k)

    def compute(rows, e_loc, c_out0):
        def center(ci, _):
            cl = e_loc // k + ci
            for half in range(2):
                bcs = [bc_all[cl, pl.ds((half * 8 + s) * 16, 16)]
                       for s in range(8)]

                def edges(j0, acc):
                    for ju in range(unroll):
                        r = ci * k + j0 + ju
                        dvec = jnp.full(
                            (16,),
                            dist_all[pl.ds(e_loc + r, 16)][0], jnp.float32)
                        for s in range(8):
                            t = (rows[r, pl.ds((half * 8 + s) * 16, 16)]
                                 + bcs[s] + dvec * ws[half * 8 + s])
                            acc[s] = acc[s] + jnp.maximum(t, jnp.float32(0.0))
                    return acc

                acc = [jnp.zeros((16,), jnp.float32) for _ in range(8)]
                if k <= 8:
                    acc = edges(0, acc)
                else:
                    acc = lax.fori_loop(
                        0, k // unroll,
                        lambda p, a: edges(p * unroll, a), acc)
                for s in range(8):
                    s_v[ci, pl.ds((half * 8 + s) * 16, 16)] = acc[s] * inv
            return _

        lax.fori_loop(0, nc, center, None)
        pltpu.sync_copy(s_v.at[pl.ds(0, nc)], s_hbm.at[pl.ds(c_out0, nc)])

    issue(rows_a, sem_a, 0)

    def pair(p, _):
        e_cur = p * 256
        issue(rows_b, sem_b, e_cur + 128)
        wait(rows_a, sem_a)
        compute(rows_a, e_cur, wid * cw + 2 * p * nc)

        @pl.when(p + 1 < nblk // 2)
        def _issue_next():
            issue(rows_a, sem_a, e_cur + 256)

        wait(rows_b, sem_b)
        compute(rows_b, e_cur + 128, wid * cw + (2 * p + 1) * nc)
        return _

    lax.fori_loop(0, nblk // 2, pair, None)


def _msg(a, bc, idx, dist, w, is_grid):
    rows_out = NG if is_grid else N
    ew = _EW_G if is_grid else _EW_A
    cw = _CW_G if is_grid else _CW_A
    mesh = plsc.VectorSubcoreMesh(core_axis_name="c", subcore_axis_name="s")
    f = pl.kernel(
        functools.partial(_sc_msg_body, is_grid),
        out_type=jax.ShapeDtypeStruct((rows_out, HIDDEN_DIM), jnp.float32),
        mesh=mesh,
        scratch_types=[
            pltpu.VMEM((HIDDEN_DIM,), jnp.float32),
            pltpu.VMEM((ew,), jnp.int32),
            pltpu.VMEM((ew + 16,), jnp.float32),
            pltpu.VMEM((cw, HIDDEN_DIM), jnp.float32),
            pltpu.VMEM((_EPB_A, HIDDEN_DIM), jnp.float32),
            pltpu.VMEM((128, HIDDEN_DIM), jnp.float32),
            pltpu.VMEM((128, HIDDEN_DIM), jnp.float32),
            pltpu.SemaphoreType.DMA,
            pltpu.SemaphoreType.DMA,
        ],
    )
    return f(a, bc, idx, dist, w)


def kernel(pos, x, batch, W1_0, b1_0, W2_0, b2_0, g_0, be_0,
           W1_1, b1_1, W2_1, b2_1, g_1, be_1):
    f32 = jnp.float32
    pos = pos.astype(f32)

    p2 = jnp.sum(pos * pos, axis=1)
    p2col = p2.reshape(N, 1)
    p2row = p2.reshape(1, N)

    g1d = jnp.linspace(-1.0, 1.0, GRID_SIZE)
    mx, my, mz = jnp.meshgrid(g1d, g1d, g1d, indexing='ij')
    gc = jnp.stack([mx, my, mz], axis=-1).reshape(-1, 3).astype(f32)
    g2col = jnp.sum(gc * gc, axis=1).reshape(N_GRID, 1)

    x_col = x.astype(jnp.int32).reshape(N, 1)

    b1r_0 = b1_0.reshape(1, HIDDEN_DIM)
    A0, Bc0_at = _pre0(x_col, W1_0[:CODE_DIM], W1_0[CODE_DIM:2 * CODE_DIM],
                       b1r_0)
    Bc0 = jnp.concatenate(
        [Bc0_at, jnp.broadcast_to(b1r_0, (NG, HIDDEN_DIM))], axis=0)

    idxG, distG = _knn_grid(gc, pos, g2col, p2row)
    S0g = _msg(A0, Bc0, idxG.reshape(-1), distG.reshape(-1),
               W1_0[2 * CODE_DIM], True)
    idxA, distA = _knn_atoms(pos, p2col, p2row)
    S0a = _msg(A0, Bc0, idxA.reshape(-1), distA.reshape(-1),
               W1_0[2 * CODE_DIM], False)

    h1, A1, Bc1 = _fuse(S0a, S0g, W2_0, b2_0.reshape(1, CODE_DIM), x_col,
                        g_0.reshape(1, CODE_DIM), be_0.reshape(1, CODE_DIM),
                        W1_1[:CODE_DIM], W1_1[CODE_DIM:2 * CODE_DIM],
                        b1_1.reshape(1, HIDDEN_DIM))

    S1 = _msg(A1, Bc1, idxG.reshape(-1), distG.reshape(-1),
              W1_1[2 * CODE_DIM], True)
    hg = _post1(S1, W2_1, b2_1.reshape(1, CODE_DIM), h1,
                g_1.reshape(1, CODE_DIM), be_1.reshape(1, CODE_DIM))
    return hg.reshape(B, N_GRID, CODE_DIM)
